# baseline (speedup 1.0000x reference)
import jax
import jax.numpy as jnp
from jax.experimental import pallas as pl
from jax.experimental.pallas import tpu as pltpu

_EPS = 1e-6
_CHUNK = 64
_TBLK = 2048


def _mom_kernel(x_ref, p_ref, o_ref, a1_ref, a2_ref, a3_ref, a4_ref, ap_ref):
    Tb = x_ref.shape[1]
    C = x_ref.shape[2]
    nck = Tb // _CHUNK
    g = _CHUNK // 8
    j = pl.program_id(1)
    nt = pl.num_programs(1)
    pv = p_ref[0]

    a1 = jnp.zeros((8, C), jnp.float32)
    a2 = jnp.zeros((8, C), jnp.float32)
    a3 = jnp.zeros((8, C), jnp.float32)
    a4 = jnp.zeros((8, C), jnp.float32)
    for k in range(nck):
        xc = x_ref[0, k * _CHUNK:(k + 1) * _CHUNK, :]
        x2 = xc * xc
        x3 = x2 * xc
        x4 = x2 * x2
        a1 = a1 + jnp.sum(xc.reshape(g, 8, C), axis=0)
        a2 = a2 + jnp.sum(x2.reshape(g, 8, C), axis=0)
        a3 = a3 + jnp.sum(x3.reshape(g, 8, C), axis=0)
        a4 = a4 + jnp.sum(x4.reshape(g, 8, C), axis=0)

    @pl.when(j == 0)
    def _():
        a1_ref[...] = a1
        a2_ref[...] = a2
        a3_ref[...] = a3
        a4_ref[...] = a4

    @pl.when(j != 0)
    def _():
        a1_ref[...] += a1
        a2_ref[...] += a2
        a3_ref[...] += a3
        a4_ref[...] += a4

    @pl.when(pv != 1.0)
    def _():
        ap = jnp.zeros((8, C), jnp.float32)
        for k in range(nck):
            xc = x_ref[0, k * _CHUNK:(k + 1) * _CHUNK, :]
            xp = jnp.exp2(pv * jnp.log2(xc))
            ap = ap + jnp.sum(xp.reshape(g, 8, C), axis=0)

        @pl.when(j == 0)
        def _():
            ap_ref[...] = ap

        @pl.when(j != 0)
        def _():
            ap_ref[...] += ap

    @pl.when(j == nt - 1)
    def _():
        inv_n = 1.0 / (Tb * nt)
        m1 = jnp.sum(a1_ref[...], axis=0, keepdims=True) * inv_n
        m2 = jnp.sum(a2_ref[...], axis=0, keepdims=True) * inv_n
        m3 = jnp.sum(a3_ref[...], axis=0, keepdims=True) * inv_n
        m4 = jnp.sum(a4_ref[...], axis=0, keepdims=True) * inv_n
        var = m2 - m1 * m1
        m1sq = m1 * m1
        cm3 = m3 - 3.0 * m1 * m2 + 2.0 * m1 * m1sq
        cm4 = m4 - 4.0 * m1 * m3 + 6.0 * m1sq * m2 - 3.0 * m1sq * m1sq
        v_eps = var + _EPS
        std = jnp.sqrt(v_eps)
        o_ref[0, 1:2, :] = var
        o_ref[0, 2:3, :] = cm3 / (v_eps * std)
        o_ref[0, 3:4, :] = cm4 / (v_eps * v_eps)

        @pl.when(pv == 1.0)
        def _():
            o_ref[0, 0:1, :] = m1

        @pl.when(pv != 1.0)
        def _():
            o_ref[0, 0:1, :] = jnp.sum(ap_ref[...], axis=0, keepdims=True) * inv_n


def _moments(x, p, *, interpret=False):
    B, T, C = x.shape
    tb = _TBLK if T % _TBLK == 0 else T
    nt = T // tb
    out3 = pl.pallas_call(
        _mom_kernel,
        out_shape=jax.ShapeDtypeStruct((B, 4, C), jnp.float32),
        grid=(B, nt),
        in_specs=[
            pl.BlockSpec((1, tb, C), lambda b, j: (b, j, 0)),
            pl.BlockSpec(memory_space=pltpu.SMEM),
        ],
        out_specs=pl.BlockSpec((1, 4, C), lambda b, j: (b, 0, 0)),
        scratch_shapes=[
            pltpu.VMEM((8, C), jnp.float32),
            pltpu.VMEM((8, C), jnp.float32),
            pltpu.VMEM((8, C), jnp.float32),
            pltpu.VMEM((8, C), jnp.float32),
            pltpu.VMEM((8, C), jnp.float32),
        ],
        compiler_params=pltpu.CompilerParams(
            dimension_semantics=("parallel", "arbitrary"),
        ),
        name="mixed_hom_moments",
        interpret=interpret,
    )(x, p)
    return out3.reshape(B, 4 * C)


def kernel(x, p):
    return _moments(x.astype(jnp.float32), p.astype(jnp.float32))


# speedup vs baseline: 1.3989x; 1.3989x over previous
import jax
import jax.numpy as jnp
from jax.experimental import pallas as pl
from jax.experimental.pallas import tpu as pltpu

_EPS = 1e-6
_CHUNK = 128


def _mom_kernel(x_ref, p_ref, o_ref):
    T = x_ref.shape[1]
    C = x_ref.shape[2]
    z = jnp.zeros((8, C), jnp.float32)

    def _tbody(k, accs):
        a1, a2, a3, a4 = accs
        xt = x_ref[0, pl.ds(k * _CHUNK, _CHUNK), :]
        x2 = xt * xt
        x3 = x2 * xt
        x4 = x2 * x2
        g = _CHUNK // 8
        a1 = a1 + jnp.sum(xt.reshape(g, 8, C), axis=0)
        a2 = a2 + jnp.sum(x2.reshape(g, 8, C), axis=0)
        a3 = a3 + jnp.sum(x3.reshape(g, 8, C), axis=0)
        a4 = a4 + jnp.sum(x4.reshape(g, 8, C), axis=0)
        return (a1, a2, a3, a4)

    a1, a2, a3, a4 = jax.lax.fori_loop(0, T // _CHUNK, _tbody, (z, z, z, z))
    inv_n = 1.0 / T
    m1 = jnp.sum(a1, axis=0, keepdims=True) * inv_n
    m2 = jnp.sum(a2, axis=0, keepdims=True) * inv_n
    m3 = jnp.sum(a3, axis=0, keepdims=True) * inv_n
    m4 = jnp.sum(a4, axis=0, keepdims=True) * inv_n
    var = m2 - m1 * m1
    m1sq = m1 * m1
    cm3 = m3 - 3.0 * m1 * m2 + 2.0 * m1 * m1sq
    cm4 = m4 - 4.0 * m1 * m3 + 6.0 * m1sq * m2 - 3.0 * m1sq * m1sq
    v_eps = var + _EPS
    std = jnp.sqrt(v_eps)
    skew = cm3 / (v_eps * std)
    kurt = cm4 / (v_eps * v_eps)
    o_ref[0, 1:2, :] = var
    o_ref[0, 2:3, :] = skew
    o_ref[0, 3:4, :] = kurt

    pv = p_ref[0]

    @pl.when(pv == 1.0)
    def _():
        o_ref[0, 0:1, :] = m1

    @pl.when(pv != 1.0)
    def _():
        def _pbody(k, ap):
            xt = x_ref[0, pl.ds(k * _CHUNK, _CHUNK), :]
            xp = jnp.exp2(pv * jnp.log2(xt))
            g = _CHUNK // 8
            return ap + jnp.sum(xp.reshape(g, 8, C), axis=0)

        ap = jax.lax.fori_loop(0, T // _CHUNK, _pbody, z)
        o_ref[0, 0:1, :] = jnp.sum(ap, axis=0, keepdims=True) * inv_n


def _moments(x, p, *, interpret=False):
    B, T, C = x.shape
    out3 = pl.pallas_call(
        _mom_kernel,
        out_shape=jax.ShapeDtypeStruct((B, 4, C), jnp.float32),
        grid=(B,),
        in_specs=[
            pl.BlockSpec((1, T, C), lambda b: (b, 0, 0)),
            pl.BlockSpec(memory_space=pltpu.SMEM),
        ],
        out_specs=pl.BlockSpec((1, 4, C), lambda b: (b, 0, 0)),
        compiler_params=pltpu.CompilerParams(
            dimension_semantics=("parallel",),
        ),
        name="mixed_hom_moments",
        interpret=interpret,
    )(x, p)
    return out3.reshape(B, 4 * C)


def kernel(x, p):
    return _moments(x.astype(jnp.float32), p.astype(jnp.float32))


# speedup vs baseline: 1.5541x; 1.1110x over previous
import jax
import jax.numpy as jnp
from jax.experimental import pallas as pl
from jax.experimental.pallas import tpu as pltpu

_EPS = 1e-6
_CHUNK = 32


def _mom_kernel(x_ref, p_ref, o_ref):
    T = x_ref.shape[1]
    C = x_ref.shape[2]
    g = _CHUNK // 8
    a1 = jnp.zeros((8, C), jnp.float32)
    a2 = jnp.zeros((8, C), jnp.float32)
    a3 = jnp.zeros((8, C), jnp.float32)
    a4 = jnp.zeros((8, C), jnp.float32)
    for k in range(T // _CHUNK):
        xc = x_ref[0, k * _CHUNK:(k + 1) * _CHUNK, :]
        x2 = xc * xc
        x3 = x2 * xc
        x4 = x2 * x2
        a1 = a1 + jnp.sum(xc.reshape(g, 8, C), axis=0)
        a2 = a2 + jnp.sum(x2.reshape(g, 8, C), axis=0)
        a3 = a3 + jnp.sum(x3.reshape(g, 8, C), axis=0)
        a4 = a4 + jnp.sum(x4.reshape(g, 8, C), axis=0)
    inv_n = 1.0 / T
    m1 = jnp.sum(a1, axis=0, keepdims=True) * inv_n
    m2 = jnp.sum(a2, axis=0, keepdims=True) * inv_n
    m3 = jnp.sum(a3, axis=0, keepdims=True) * inv_n
    m4 = jnp.sum(a4, axis=0, keepdims=True) * inv_n
    var = m2 - m1 * m1
    m1sq = m1 * m1
    cm3 = m3 - 3.0 * m1 * m2 + 2.0 * m1 * m1sq
    cm4 = m4 - 4.0 * m1 * m3 + 6.0 * m1sq * m2 - 3.0 * m1sq * m1sq
    v_eps = var + _EPS
    std = jnp.sqrt(v_eps)
    skew = cm3 / (v_eps * std)
    kurt = cm4 / (v_eps * v_eps)
    o_ref[0, 1:2, :] = var
    o_ref[0, 2:3, :] = skew
    o_ref[0, 3:4, :] = kurt

    pv = p_ref[0]

    @pl.when(pv == 1.0)
    def _():
        o_ref[0, 0:1, :] = m1

    @pl.when(pv != 1.0)
    def _():
        def _pbody(k, ap):
            xt = x_ref[0, pl.ds(k * _CHUNK, _CHUNK), :]
            xp = jnp.exp2(pv * jnp.log2(xt))
            return ap + jnp.sum(xp.reshape(g, 8, C), axis=0)

        ap = jax.lax.fori_loop(0, T // _CHUNK, _pbody,
                               jnp.zeros((8, C), jnp.float32))
        o_ref[0, 0:1, :] = jnp.sum(ap, axis=0, keepdims=True) * inv_n


def _moments(x, p, *, interpret=False):
    B, T, C = x.shape
    out3 = pl.pallas_call(
        _mom_kernel,
        out_shape=jax.ShapeDtypeStruct((B, 4, C), jnp.float32),
        grid=(B,),
        in_specs=[
            pl.BlockSpec((1, T, C), lambda b: (b, 0, 0)),
            pl.BlockSpec(memory_space=pltpu.SMEM),
        ],
        out_specs=pl.BlockSpec((1, 4, C), lambda b: (b, 0, 0)),
        compiler_params=pltpu.CompilerParams(
            dimension_semantics=("parallel",),
        ),
        name="mixed_hom_moments",
        interpret=interpret,
    )(x, p)
    return out3.reshape(B, 4 * C)


def kernel(x, p):
    return _moments(x.astype(jnp.float32), p.astype(jnp.float32))


# speedup vs baseline: 1.6143x; 1.0387x over previous
import jax
import jax.numpy as jnp
from jax.experimental import pallas as pl
from jax.experimental.pallas import tpu as pltpu

_EPS = 1e-6
_CHUNK = 32


def _mom_kernel(x_hbm, p_ref, o_ref, buf, sems):
    B = x_hbm.shape[0]
    T = x_hbm.shape[1]
    C = x_hbm.shape[2]
    g = _CHUNK // 8
    pv = p_ref[0]

    def _start(b, slot):
        pltpu.make_async_copy(x_hbm.at[b], buf.at[slot], sems.at[slot]).start()

    def _wait(slot):
        pltpu.make_async_copy(buf.at[slot], buf.at[slot], sems.at[slot]).wait()

    _start(0, 0)

    def _body(b, _):
        cur = jax.lax.rem(b, 2)
        nxt = jax.lax.rem(b + 1, 2)

        @pl.when(b + 1 < B)
        def _():
            _start(b + 1, nxt)

        _wait(cur)
        x_ref = buf.at[cur]

        a1 = jnp.zeros((8, C), jnp.float32)
        a2 = jnp.zeros((8, C), jnp.float32)
        a3 = jnp.zeros((8, C), jnp.float32)
        a4 = jnp.zeros((8, C), jnp.float32)
        for k in range(T // _CHUNK):
            xc = x_ref[k * _CHUNK:(k + 1) * _CHUNK, :]
            x2 = xc * xc
            x3 = x2 * xc
            x4 = x2 * x2
            a1 = a1 + jnp.sum(xc.reshape(g, 8, C), axis=0)
            a2 = a2 + jnp.sum(x2.reshape(g, 8, C), axis=0)
            a3 = a3 + jnp.sum(x3.reshape(g, 8, C), axis=0)
            a4 = a4 + jnp.sum(x4.reshape(g, 8, C), axis=0)
        inv_n = 1.0 / T
        m1 = jnp.sum(a1, axis=0, keepdims=True) * inv_n
        m2 = jnp.sum(a2, axis=0, keepdims=True) * inv_n
        m3 = jnp.sum(a3, axis=0, keepdims=True) * inv_n
        m4 = jnp.sum(a4, axis=0, keepdims=True) * inv_n
        var = m2 - m1 * m1
        m1sq = m1 * m1
        cm3 = m3 - 3.0 * m1 * m2 + 2.0 * m1 * m1sq
        cm4 = m4 - 4.0 * m1 * m3 + 6.0 * m1sq * m2 - 3.0 * m1sq * m1sq
        v_eps = var + _EPS
        std = jnp.sqrt(v_eps)
        skew = cm3 / (v_eps * std)
        kurt = cm4 / (v_eps * v_eps)
        rows = jnp.concatenate([m1, var, skew, kurt], axis=0)
        o_ref[pl.ds(b, 1), :, :] = rows.reshape(1, 4, C)

        @pl.when(pv != 1.0)
        def _():
            def _pbody(k, ap):
                xt = x_ref[pl.ds(k * _CHUNK, _CHUNK), :]
                xp = jnp.exp2(pv * jnp.log2(xt))
                return ap + jnp.sum(xp.reshape(g, 8, C), axis=0)

            ap = jax.lax.fori_loop(0, T // _CHUNK, _pbody,
                                   jnp.zeros((8, C), jnp.float32))
            gp = jnp.sum(ap, axis=0, keepdims=True) * inv_n
            o_ref[pl.ds(b, 1), 0:1, :] = gp.reshape(1, 1, C)

        return ()

    jax.lax.fori_loop(0, B, _body, ())


def _moments(x, p, *, interpret=False):
    B, T, C = x.shape
    out3 = pl.pallas_call(
        _mom_kernel,
        out_shape=jax.ShapeDtypeStruct((B, 4, C), jnp.float32),
        in_specs=[
            pl.BlockSpec(memory_space=pl.ANY),
            pl.BlockSpec(memory_space=pltpu.SMEM),
        ],
        out_specs=pl.BlockSpec(memory_space=pltpu.VMEM),
        scratch_shapes=[
            pltpu.VMEM((2, T, C), jnp.float32),
            pltpu.SemaphoreType.DMA((2,)),
        ],
        name="mixed_hom_moments",
        interpret=interpret,
    )(x, p)
    return out3.reshape(B, 4 * C)


def kernel(x, p):
    if x.dtype != jnp.float32:
        x = x.astype(jnp.float32)
    if p.dtype != jnp.float32:
        p = p.astype(jnp.float32)
    return _moments(x, p)


# speedup vs baseline: 1.7485x; 1.0831x over previous
import jax
import jax.numpy as jnp
from jax.experimental import pallas as pl
from jax.experimental.pallas import tpu as pltpu

_EPS = 1e-6
_CHUNK = 32
_GB = 2


def _mom_kernel(x_hbm, p_ref, o_ref, buf, sems):
    B = x_hbm.shape[0]
    T = x_hbm.shape[1]
    C = x_hbm.shape[2]
    g = _CHUNK // 8
    pv = p_ref[0]
    nb = B // _GB

    def _start(i, slot):
        pltpu.make_async_copy(x_hbm.at[pl.ds(i * _GB, _GB)], buf.at[slot],
                              sems.at[slot]).start()

    def _wait(slot):
        pltpu.make_async_copy(buf.at[slot], buf.at[slot], sems.at[slot]).wait()

    _start(0, 0)

    def _body(i, _):
        cur = jax.lax.rem(i, 2)
        nxt = jax.lax.rem(i + 1, 2)

        @pl.when(i + 1 < nb)
        def _():
            _start(i + 1, nxt)

        _wait(cur)

        for b2 in range(_GB):
            x_ref = buf.at[cur, b2]
            b = i * _GB + b2

            a1 = jnp.zeros((8, C), jnp.float32)
            a2 = jnp.zeros((8, C), jnp.float32)
            a3 = jnp.zeros((8, C), jnp.float32)
            a4 = jnp.zeros((8, C), jnp.float32)
            for k in range(T // _CHUNK):
                xc = x_ref[k * _CHUNK:(k + 1) * _CHUNK, :]
                x2 = xc * xc
                x3 = x2 * xc
                x4 = x2 * x2
                a1 = a1 + jnp.sum(xc.reshape(g, 8, C), axis=0)
                a2 = a2 + jnp.sum(x2.reshape(g, 8, C), axis=0)
                a3 = a3 + jnp.sum(x3.reshape(g, 8, C), axis=0)
                a4 = a4 + jnp.sum(x4.reshape(g, 8, C), axis=0)
            inv_n = 1.0 / T
            m1 = jnp.sum(a1, axis=0, keepdims=True) * inv_n
            m2 = jnp.sum(a2, axis=0, keepdims=True) * inv_n
            m3 = jnp.sum(a3, axis=0, keepdims=True) * inv_n
            m4 = jnp.sum(a4, axis=0, keepdims=True) * inv_n
            var = m2 - m1 * m1
            m1sq = m1 * m1
            cm3 = m3 - 3.0 * m1 * m2 + 2.0 * m1 * m1sq
            cm4 = m4 - 4.0 * m1 * m3 + 6.0 * m1sq * m2 - 3.0 * m1sq * m1sq
            v_eps = var + _EPS
            std = jnp.sqrt(v_eps)
            skew = cm3 / (v_eps * std)
            kurt = cm4 / (v_eps * v_eps)
            rows = jnp.concatenate([m1, var, skew, kurt], axis=0)
            o_ref[pl.ds(b, 1), :, :] = rows.reshape(1, 4, C)

            @pl.when(pv != 1.0)
            def _():
                def _pbody(k, ap):
                    xt = x_ref[pl.ds(k * _CHUNK, _CHUNK), :]
                    xp = jnp.exp2(pv * jnp.log2(xt))
                    return ap + jnp.sum(xp.reshape(g, 8, C), axis=0)

                ap = jax.lax.fori_loop(0, T // _CHUNK, _pbody,
                                       jnp.zeros((8, C), jnp.float32))
                gp = jnp.sum(ap, axis=0, keepdims=True) * inv_n
                o_ref[pl.ds(b, 1), 0:1, :] = gp.reshape(1, 1, C)

        return ()

    jax.lax.fori_loop(0, nb, _body, ())


def _moments(x, p, *, interpret=False):
    B, T, C = x.shape
    out3 = pl.pallas_call(
        _mom_kernel,
        out_shape=jax.ShapeDtypeStruct((B, 4, C), jnp.float32),
        in_specs=[
            pl.BlockSpec(memory_space=pl.ANY),
            pl.BlockSpec(memory_space=pltpu.SMEM),
        ],
        out_specs=pl.BlockSpec(memory_space=pltpu.VMEM),
        scratch_shapes=[
            pltpu.VMEM((2, _GB, T, C), jnp.float32),
            pltpu.SemaphoreType.DMA((2,)),
        ],
        compiler_params=pltpu.CompilerParams(
            vmem_limit_bytes=48 * 1024 * 1024,
        ),
        name="mixed_hom_moments",
        interpret=interpret,
    )(x, p)
    return out3.reshape(B, 4 * C)


def kernel(x, p):
    if x.dtype != jnp.float32:
        x = x.astype(jnp.float32)
    if p.dtype != jnp.float32:
        p = p.astype(jnp.float32)
    return _moments(x, p)


# speedup vs baseline: 1.7522x; 1.0021x over previous
import jax
import jax.numpy as jnp
from jax.experimental import pallas as pl
from jax.experimental.pallas import tpu as pltpu

_EPS = 1e-6
_CHUNK = 32
_GB = 2


def _mom_kernel(x_hbm, p_ref, o_ref, buf, sems):
    B = x_hbm.shape[0]
    T = x_hbm.shape[1]
    C = x_hbm.shape[2]
    g = _CHUNK // 8
    pv = p_ref[0]
    nb = B // _GB

    def _start(i, slot):
        pltpu.make_async_copy(x_hbm.at[pl.ds(i * _GB, _GB)], buf.at[slot],
                              sems.at[slot]).start()

    def _wait(slot):
        pltpu.make_async_copy(buf.at[slot], buf.at[slot], sems.at[slot]).wait()

    _start(0, 0)
    if nb > 1:
        _start(1, 1)

    def _body(i, _):
        cur = jax.lax.rem(i, 3)

        @pl.when(i + 2 < nb)
        def _():
            _start(i + 2, jax.lax.rem(i + 2, 3))

        _wait(cur)

        for b2 in range(_GB):
            x_ref = buf.at[cur, b2]
            b = i * _GB + b2

            a1 = jnp.zeros((8, C), jnp.float32)
            a2 = jnp.zeros((8, C), jnp.float32)
            a3 = jnp.zeros((8, C), jnp.float32)
            a4 = jnp.zeros((8, C), jnp.float32)
            for k in range(T // _CHUNK):
                xc = x_ref[k * _CHUNK:(k + 1) * _CHUNK, :]
                x2 = xc * xc
                x3 = x2 * xc
                x4 = x2 * x2
                a1 = a1 + jnp.sum(xc.reshape(g, 8, C), axis=0)
                a2 = a2 + jnp.sum(x2.reshape(g, 8, C), axis=0)
                a3 = a3 + jnp.sum(x3.reshape(g, 8, C), axis=0)
                a4 = a4 + jnp.sum(x4.reshape(g, 8, C), axis=0)
            inv_n = 1.0 / T
            m1 = jnp.sum(a1, axis=0, keepdims=True) * inv_n
            m2 = jnp.sum(a2, axis=0, keepdims=True) * inv_n
            m3 = jnp.sum(a3, axis=0, keepdims=True) * inv_n
            m4 = jnp.sum(a4, axis=0, keepdims=True) * inv_n
            var = m2 - m1 * m1
            m1sq = m1 * m1
            cm3 = m3 - 3.0 * m1 * m2 + 2.0 * m1 * m1sq
            cm4 = m4 - 4.0 * m1 * m3 + 6.0 * m1sq * m2 - 3.0 * m1sq * m1sq
            v_eps = var + _EPS
            std = jnp.sqrt(v_eps)
            skew = cm3 / (v_eps * std)
            kurt = cm4 / (v_eps * v_eps)
            rows = jnp.concatenate([m1, var, skew, kurt], axis=0)
            o_ref[pl.ds(b, 1), :, :] = rows.reshape(1, 4, C)

            @pl.when(pv != 1.0)
            def _():
                def _pbody(k, ap):
                    xt = x_ref[pl.ds(k * _CHUNK, _CHUNK), :]
                    xp = jnp.exp2(pv * jnp.log2(xt))
                    return ap + jnp.sum(xp.reshape(g, 8, C), axis=0)

                ap = jax.lax.fori_loop(0, T // _CHUNK, _pbody,
                                       jnp.zeros((8, C), jnp.float32))
                gp = jnp.sum(ap, axis=0, keepdims=True) * inv_n
                o_ref[pl.ds(b, 1), 0:1, :] = gp.reshape(1, 1, C)

        return ()

    jax.lax.fori_loop(0, nb, _body, ())


def _moments(x, p, *, interpret=False):
    B, T, C = x.shape
    out3 = pl.pallas_call(
        _mom_kernel,
        out_shape=jax.ShapeDtypeStruct((B, 4, C), jnp.float32),
        in_specs=[
            pl.BlockSpec(memory_space=pl.ANY),
            pl.BlockSpec(memory_space=pltpu.SMEM),
        ],
        out_specs=pl.BlockSpec(memory_space=pltpu.VMEM),
        scratch_shapes=[
            pltpu.VMEM((3, _GB, T, C), jnp.float32),
            pltpu.SemaphoreType.DMA((3,)),
        ],
        compiler_params=pltpu.CompilerParams(
            vmem_limit_bytes=56 * 1024 * 1024,
        ),
        name="mixed_hom_moments",
        interpret=interpret,
    )(x, p)
    return out3.reshape(B, 4 * C)


def kernel(x, p):
    if x.dtype != jnp.float32:
        x = x.astype(jnp.float32)
    if p.dtype != jnp.float32:
        p = p.astype(jnp.float32)
    return _moments(x, p)


# speedup vs baseline: 1.7906x; 1.0219x over previous
import jax
import jax.numpy as jnp
from jax.experimental import pallas as pl
from jax.experimental.pallas import tpu as pltpu

_EPS = 1e-6
_CHUNK = 32
_GB = 2


def _mom_kernel(x_hbm, p_ref, o_ref, buf, sems):
    B = x_hbm.shape[0]
    T = x_hbm.shape[1]
    C = x_hbm.shape[2]
    g = _CHUNK // 8
    pv = p_ref[0]
    nb = B // _GB

    def _start(i, slot):
        for h in range(_GB):
            pltpu.make_async_copy(x_hbm.at[i * _GB + h], buf.at[slot, h],
                                  sems.at[slot, h]).start()

    def _wait(slot, half):
        pltpu.make_async_copy(buf.at[slot, half], buf.at[slot, half],
                              sems.at[slot, half]).wait()

    _start(0, 0)
    if nb > 1:
        _start(1, 1)

    def _body(i, _):
        cur = jax.lax.rem(i, 3)

        @pl.when(i + 2 < nb)
        def _():
            _start(i + 2, jax.lax.rem(i + 2, 3))

        for b2 in range(_GB):
            _wait(cur, b2)
            x_ref = buf.at[cur, b2]
            b = i * _GB + b2

            a1 = jnp.zeros((8, C), jnp.float32)
            a2 = jnp.zeros((8, C), jnp.float32)
            a3 = jnp.zeros((8, C), jnp.float32)
            a4 = jnp.zeros((8, C), jnp.float32)
            for k in range(T // _CHUNK):
                xc = x_ref[k * _CHUNK:(k + 1) * _CHUNK, :]
                x2 = xc * xc
                x3 = x2 * xc
                x4 = x2 * x2
                a1 = a1 + jnp.sum(xc.reshape(g, 8, C), axis=0)
                a2 = a2 + jnp.sum(x2.reshape(g, 8, C), axis=0)
                a3 = a3 + jnp.sum(x3.reshape(g, 8, C), axis=0)
                a4 = a4 + jnp.sum(x4.reshape(g, 8, C), axis=0)
            inv_n = 1.0 / T
            m1 = jnp.sum(a1, axis=0, keepdims=True) * inv_n
            m2 = jnp.sum(a2, axis=0, keepdims=True) * inv_n
            m3 = jnp.sum(a3, axis=0, keepdims=True) * inv_n
            m4 = jnp.sum(a4, axis=0, keepdims=True) * inv_n
            var = m2 - m1 * m1
            m1sq = m1 * m1
            cm3 = m3 - 3.0 * m1 * m2 + 2.0 * m1 * m1sq
            cm4 = m4 - 4.0 * m1 * m3 + 6.0 * m1sq * m2 - 3.0 * m1sq * m1sq
            v_eps = var + _EPS
            std = jnp.sqrt(v_eps)
            skew = cm3 / (v_eps * std)
            kurt = cm4 / (v_eps * v_eps)
            rows = jnp.concatenate([m1, var, skew, kurt], axis=0)
            o_ref[pl.ds(b, 1), :, :] = rows.reshape(1, 4, C)

            @pl.when(pv != 1.0)
            def _():
                def _pbody(k, ap):
                    xt = x_ref[pl.ds(k * _CHUNK, _CHUNK), :]
                    xp = jnp.exp2(pv * jnp.log2(xt))
                    return ap + jnp.sum(xp.reshape(g, 8, C), axis=0)

                ap = jax.lax.fori_loop(0, T // _CHUNK, _pbody,
                                       jnp.zeros((8, C), jnp.float32))
                gp = jnp.sum(ap, axis=0, keepdims=True) * inv_n
                o_ref[pl.ds(b, 1), 0:1, :] = gp.reshape(1, 1, C)

        return ()

    jax.lax.fori_loop(0, nb, _body, ())


def _moments(x, p, *, interpret=False):
    B, T, C = x.shape
    out3 = pl.pallas_call(
        _mom_kernel,
        out_shape=jax.ShapeDtypeStruct((B, 4, C), jnp.float32),
        in_specs=[
            pl.BlockSpec(memory_space=pl.ANY),
            pl.BlockSpec(memory_space=pltpu.SMEM),
        ],
        out_specs=pl.BlockSpec(memory_space=pltpu.VMEM),
        scratch_shapes=[
            pltpu.VMEM((3, _GB, T, C), jnp.float32),
            pltpu.SemaphoreType.DMA((3, _GB)),
        ],
        compiler_params=pltpu.CompilerParams(
            vmem_limit_bytes=56 * 1024 * 1024,
        ),
        name="mixed_hom_moments",
        interpret=interpret,
    )(x, p)
    return out3.reshape(B, 4 * C)


def kernel(x, p):
    if x.dtype != jnp.float32:
        x = x.astype(jnp.float32)
    if p.dtype != jnp.float32:
        p = p.astype(jnp.float32)
    return _moments(x, p)
